# baseline (speedup 1.0000x reference)
import functools

import jax
import jax.numpy as jnp
from jax import lax
from jax.experimental import pallas as pl
from jax.experimental.pallas import tpu as pltpu
from jax.experimental.pallas import tpu_sc as plsc

N = 10000
NP = 10240
D = 128
E = 320000

NC = 2
NS = 16
L = 16
NW = NC * NS

CHUNK = 128
EPW = 10240
E_PAD = NW * EPW
NCHUNK = EPW // CHUNK
IDX_BLK = 8
NG = CHUNK // L
RPT = NP // NS

ROW_BLK = 1024
GRID = NP // ROW_BLK


def _sc_edge_body(ft_hbm, src_hbm, dst_hbm, a_hbm, z2_hbm, z1_hbm,
                  u_out, den_out,
                  src_idx, dst_idx, rows_s, rows_d, ex_v, a_v,
                  u_sh, den_sh, sem1, sem2):
    c = lax.axis_index("c")
    s = lax.axis_index("s")
    w = c * NS + s

    pltpu.sync_copy(z2_hbm.at[pl.ds(s * RPT, RPT)],
                    u_sh.at[pl.ds(s * RPT, RPT)])
    pltpu.sync_copy(z1_hbm.at[pl.ds(s * RPT, RPT)],
                    den_sh.at[pl.ds(s * RPT, RPT)])
    pltpu.sync_copy(a_hbm, a_v)
    plsc.subcore_barrier()

    def block_body(jb, carry_b):
        pltpu.sync_copy(src_hbm.at[pl.ds(w * NCHUNK + jb * IDX_BLK, IDX_BLK)],
                        src_idx)
        pltpu.sync_copy(dst_hbm.at[pl.ds(w * NCHUNK + jb * IDX_BLK, IDX_BLK)],
                        dst_idx)

        def chunk_body(jj, carry):
            j = jb * IDX_BLK + jj
            g1 = pltpu.async_copy(ft_hbm.at[src_idx.at[jj]], rows_s, sem1)
            g2 = pltpu.async_copy(ft_hbm.at[dst_idx.at[jj]], rows_d, sem2)
            g1.wait()
            g2.wait()
            ebase = w * EPW + j * CHUNK
            eidx_g = [g * L + lax.iota(jnp.int32, L) for g in range(NG)]

            def dot_body(d, accs):
                dsplat = jnp.full((L,), d, jnp.int32)
                av = plsc.load_gather(a_v, [dsplat])
                out = []
                for g in range(NG):
                    cs = plsc.load_gather(rows_s, [eidx_g[g], dsplat])
                    cd = plsc.load_gather(rows_d, [eidx_g[g], dsplat])
                    t = cs + cd
                    lr = jnp.maximum(t, 0.2 * t)
                    out.append(accs[g] + av * lr)
                return tuple(out)

            accs = lax.fori_loop(0, D, dot_body,
                                 (jnp.zeros((L,), jnp.float32),) * NG,
                                 unroll=2)
            exs = []
            for g in range(NG):
                ex = jnp.exp(accs[g])
                ex = jnp.where(ebase + eidx_g[g] < E, ex, 0.0)
                exs.append(ex)
                ex_v[pl.ds(g * L, L)] = ex

            def scale_body(d, carry_d):
                dsplat = jnp.full((L,), d, jnp.int32)
                for g in range(NG):
                    cs = plsc.load_gather(rows_s, [eidx_g[g], dsplat])
                    plsc.store_scatter(rows_s, [eidx_g[g], dsplat],
                                       cs * exs[g])
                return carry_d

            lax.fori_loop(0, D, scale_body, 0, unroll=2)
            pltpu.sync_copy(rows_s, u_sh.at[dst_idx.at[jj]], add=True)
            pltpu.sync_copy(ex_v, den_sh.at[dst_idx.at[jj]], add=True)
            return carry

        lax.fori_loop(0, IDX_BLK, chunk_body, 0)
        return carry_b

    lax.fori_loop(0, NCHUNK // IDX_BLK, block_body, 0)
    plsc.subcore_barrier()
    pltpu.sync_copy(u_sh.at[pl.ds(s * RPT, RPT)],
                    u_out.at[c, pl.ds(s * RPT, RPT)])
    pltpu.sync_copy(den_sh.at[pl.ds(s * RPT, RPT)],
                    den_out.at[c, pl.ds(s * RPT, RPT)])


_sc_edge = pl.kernel(
    _sc_edge_body,
    out_type=[
        jax.ShapeDtypeStruct((NC, NP, D), jnp.float32),
        jax.ShapeDtypeStruct((NC, NP), jnp.float32),
    ],
    mesh=plsc.VectorSubcoreMesh(core_axis_name="c", subcore_axis_name="s"),
    compiler_params=pltpu.CompilerParams(needs_layout_passes=False),
    scratch_types=[
        pltpu.VMEM((IDX_BLK, CHUNK), jnp.int32),
        pltpu.VMEM((IDX_BLK, CHUNK), jnp.int32),
        pltpu.VMEM((CHUNK, D), jnp.float32),
        pltpu.VMEM((CHUNK, D), jnp.float32),
        pltpu.VMEM((CHUNK,), jnp.float32),
        pltpu.VMEM((D,), jnp.float32),
        pltpu.VMEM_SHARED((NP, D), jnp.float32),
        pltpu.VMEM_SHARED((NP,), jnp.float32),
        pltpu.SemaphoreType.DMA,
        pltpu.SemaphoreType.DMA,
    ],
)


def _mm_body(x_ref, w_ref, o_ref):
    o_ref[...] = jnp.dot(x_ref[...], w_ref[...],
                         preferred_element_type=jnp.float32)


_mm = pl.pallas_call(
    _mm_body,
    grid=(GRID,),
    in_specs=[
        pl.BlockSpec((ROW_BLK, D), lambda i: (i, 0)),
        pl.BlockSpec((D, D), lambda i: (0, 0)),
    ],
    out_specs=pl.BlockSpec((ROW_BLK, D), lambda i: (i, 0)),
    out_shape=jax.ShapeDtypeStruct((NP, D), jnp.float32),
)


def _combine_body(u_ref, den_ref, res_ref, w_ref, h_ref, ft_ref):
    u = u_ref[0] + u_ref[1]
    dsum = den_ref[:, 0:1] + den_ref[:, 1:2]
    dsafe = jnp.where(dsum == 0.0, 1.0, dsum)
    v = u / dsafe + res_ref[...]
    h = jnp.where(v > 0, v, jnp.exp(v) - 1.0)
    h_ref[...] = h
    ft_ref[...] = jnp.dot(h, w_ref[...], preferred_element_type=jnp.float32)


_combine = pl.pallas_call(
    _combine_body,
    grid=(GRID,),
    in_specs=[
        pl.BlockSpec((NC, ROW_BLK, D), lambda i: (0, i, 0)),
        pl.BlockSpec((ROW_BLK, NC), lambda i: (i, 0)),
        pl.BlockSpec((ROW_BLK, D), lambda i: (i, 0)),
        pl.BlockSpec((D, D), lambda i: (0, 0)),
    ],
    out_specs=[
        pl.BlockSpec((ROW_BLK, D), lambda i: (i, 0)),
        pl.BlockSpec((ROW_BLK, D), lambda i: (i, 0)),
    ],
    out_shape=[
        jax.ShapeDtypeStruct((NP, D), jnp.float32),
        jax.ShapeDtypeStruct((NP, D), jnp.float32),
    ],
)


def _final_body(u_ref, den_ref, res_ref, o_ref):
    i = pl.program_id(0)
    u = u_ref[0] + u_ref[1]
    dsum = den_ref[:, 0:1] + den_ref[:, 1:2]
    dsafe = jnp.where(dsum == 0.0, 1.0, dsum)
    v = u / dsafe + res_ref[...]
    h = jnp.where(v > 0, v, jnp.exp(v) - 1.0)
    part = jnp.sum(h, axis=0, keepdims=True) * (1.0 / N)

    @pl.when(i == 0)
    def _():
        o_ref[...] = jnp.zeros_like(o_ref)

    o_ref[...] += part


_final = pl.pallas_call(
    _final_body,
    grid=(GRID,),
    in_specs=[
        pl.BlockSpec((NC, ROW_BLK, D), lambda i: (0, i, 0)),
        pl.BlockSpec((ROW_BLK, NC), lambda i: (i, 0)),
        pl.BlockSpec((ROW_BLK, D), lambda i: (i, 0)),
    ],
    out_specs=pl.BlockSpec((1, D), lambda i: (0, 0)),
    out_shape=jax.ShapeDtypeStruct((1, D), jnp.float32),
)


def kernel(x, edge_index, W0, a0, W1, a1, W2, a2):
    src = edge_index[0].astype(jnp.int32)
    dst = edge_index[1].astype(jnp.int32)
    pad = jnp.zeros((E_PAD - E,), jnp.int32)
    src2d = jnp.concatenate([src, pad]).reshape(NW * NCHUNK, CHUNK)
    dst2d = jnp.concatenate([dst, pad]).reshape(NW * NCHUNK, CHUNK)
    z2 = jnp.zeros((NP, D), jnp.float32)
    z1 = jnp.zeros((NP,), jnp.float32)
    zres = jnp.zeros((NP, D), jnp.float32)
    xp = jnp.concatenate([x, jnp.zeros((NP - N, D), jnp.float32)])

    ft = _mm(xp, W0)
    u, den = _sc_edge(ft, src2d, dst2d, a0.reshape(D), z2, z1)
    h1, ft = _combine(u, den.T, zres, W1)
    u, den = _sc_edge(ft, src2d, dst2d, a1.reshape(D), z2, z1)
    h2, ft = _combine(u, den.T, h1, W2)
    u, den = _sc_edge(ft, src2d, dst2d, a2.reshape(D), z2, z1)
    return _final(u, den.T, h2)


# speedup vs baseline: 2.9089x; 2.9089x over previous
import functools

import jax
import jax.numpy as jnp
from jax import lax
from jax.experimental import pallas as pl
from jax.experimental.pallas import tpu as pltpu
from jax.experimental.pallas import tpu_sc as plsc

N = 10000
NP = 10240
D = 128
E = 320000

NC = 2
NS = 16
L = 16
NW = NC * NS

CHUNK = 128
EPW = 10240
E_PAD = NW * EPW
NCHUNK = EPW // CHUNK
IDX_BLK = 8
NG = CHUNK // L
RPT = NP // NS

ROW_BLK = 1024
GRID = NP // ROW_BLK


def _sc_edge_body(ft_hbm, src_hbm, dst_hbm, a_hbm, z2_hbm, z1_hbm,
                  u_out, den_out,
                  src_idx, dst_idx, rows_s, rows_d, ex_v, a_v,
                  u_sh, den_sh, sem1, sem2):
    c = lax.axis_index("c")
    s = lax.axis_index("s")
    w = c * NS + s

    pltpu.sync_copy(z2_hbm.at[pl.ds(s * RPT, RPT)],
                    u_sh.at[pl.ds(s * RPT, RPT)])
    pltpu.sync_copy(z1_hbm.at[pl.ds(s * RPT, RPT)],
                    den_sh.at[pl.ds(s * RPT, RPT)])
    pltpu.sync_copy(a_hbm, a_v)
    avs = [a_v[pl.ds(jd * L, L)] for jd in range(D // L)]
    lane = lax.iota(jnp.int32, L)
    lane0 = lane == 0
    splat15 = jnp.full((L,), L - 1, jnp.int32)
    plsc.subcore_barrier()

    def block_body(jb, carry_b):
        pltpu.sync_copy(src_hbm.at[pl.ds(w * NCHUNK + jb * IDX_BLK, IDX_BLK)],
                        src_idx)
        pltpu.sync_copy(dst_hbm.at[pl.ds(w * NCHUNK + jb * IDX_BLK, IDX_BLK)],
                        dst_idx)

        def chunk_body(jj, carry):
            j = jb * IDX_BLK + jj
            g1 = pltpu.async_copy(ft_hbm.at[src_idx.at[jj]], rows_s, sem1)
            g2 = pltpu.async_copy(ft_hbm.at[dst_idx.at[jj]], rows_d, sem2)
            g1.wait()
            g2.wait()
            ebase = w * EPW + j * CHUNK

            def edge_body(e, carry_e):
                srow = [rows_s[e, pl.ds(jd * L, L)] for jd in range(D // L)]
                drow = [rows_d[e, pl.ds(jd * L, L)] for jd in range(D // L)]
                acc = jnp.zeros((L,), jnp.float32)
                for jd in range(D // L):
                    t = srow[jd] + drow[jd]
                    lr = jnp.maximum(t, 0.2 * t)
                    acc = acc + avs[jd] * lr
                cum = plsc.cumsum(acc)
                tot = cum.at[splat15].get(mode="promise_in_bounds")
                ex = jnp.exp(tot)
                ex = jnp.where(ebase + e < E, ex, 0.0)
                plsc.store_scatter(ex_v, [jnp.full((L,), e, jnp.int32)], ex,
                                   mask=lane0)
                for jd in range(D // L):
                    rows_s[e, pl.ds(jd * L, L)] = srow[jd] * ex
                return carry_e

            lax.fori_loop(0, CHUNK, edge_body, 0, unroll=2)
            pltpu.sync_copy(rows_s, u_sh.at[dst_idx.at[jj]], add=True)
            pltpu.sync_copy(ex_v, den_sh.at[dst_idx.at[jj]], add=True)
            return carry

        lax.fori_loop(0, IDX_BLK, chunk_body, 0)
        return carry_b

    lax.fori_loop(0, NCHUNK // IDX_BLK, block_body, 0)
    plsc.subcore_barrier()
    pltpu.sync_copy(u_sh.at[pl.ds(s * RPT, RPT)],
                    u_out.at[c, pl.ds(s * RPT, RPT)])
    pltpu.sync_copy(den_sh.at[pl.ds(s * RPT, RPT)],
                    den_out.at[c, pl.ds(s * RPT, RPT)])


_sc_edge = pl.kernel(
    _sc_edge_body,
    out_type=[
        jax.ShapeDtypeStruct((NC, NP, D), jnp.float32),
        jax.ShapeDtypeStruct((NC, NP), jnp.float32),
    ],
    mesh=plsc.VectorSubcoreMesh(core_axis_name="c", subcore_axis_name="s"),
    compiler_params=pltpu.CompilerParams(needs_layout_passes=False),
    scratch_types=[
        pltpu.VMEM((IDX_BLK, CHUNK), jnp.int32),
        pltpu.VMEM((IDX_BLK, CHUNK), jnp.int32),
        pltpu.VMEM((CHUNK, D), jnp.float32),
        pltpu.VMEM((CHUNK, D), jnp.float32),
        pltpu.VMEM((CHUNK,), jnp.float32),
        pltpu.VMEM((D,), jnp.float32),
        pltpu.VMEM_SHARED((NP, D), jnp.float32),
        pltpu.VMEM_SHARED((NP,), jnp.float32),
        pltpu.SemaphoreType.DMA,
        pltpu.SemaphoreType.DMA,
    ],
)


def _mm_body(x_ref, w_ref, o_ref):
    o_ref[...] = jnp.dot(x_ref[...], w_ref[...],
                         preferred_element_type=jnp.float32)


_mm = pl.pallas_call(
    _mm_body,
    grid=(GRID,),
    in_specs=[
        pl.BlockSpec((ROW_BLK, D), lambda i: (i, 0)),
        pl.BlockSpec((D, D), lambda i: (0, 0)),
    ],
    out_specs=pl.BlockSpec((ROW_BLK, D), lambda i: (i, 0)),
    out_shape=jax.ShapeDtypeStruct((NP, D), jnp.float32),
)


def _combine_body(u_ref, den_ref, res_ref, w_ref, h_ref, ft_ref):
    u = u_ref[0] + u_ref[1]
    dsum = den_ref[:, 0:1] + den_ref[:, 1:2]
    dsafe = jnp.where(dsum == 0.0, 1.0, dsum)
    v = u / dsafe + res_ref[...]
    h = jnp.where(v > 0, v, jnp.exp(v) - 1.0)
    h_ref[...] = h
    ft_ref[...] = jnp.dot(h, w_ref[...], preferred_element_type=jnp.float32)


_combine = pl.pallas_call(
    _combine_body,
    grid=(GRID,),
    in_specs=[
        pl.BlockSpec((NC, ROW_BLK, D), lambda i: (0, i, 0)),
        pl.BlockSpec((ROW_BLK, NC), lambda i: (i, 0)),
        pl.BlockSpec((ROW_BLK, D), lambda i: (i, 0)),
        pl.BlockSpec((D, D), lambda i: (0, 0)),
    ],
    out_specs=[
        pl.BlockSpec((ROW_BLK, D), lambda i: (i, 0)),
        pl.BlockSpec((ROW_BLK, D), lambda i: (i, 0)),
    ],
    out_shape=[
        jax.ShapeDtypeStruct((NP, D), jnp.float32),
        jax.ShapeDtypeStruct((NP, D), jnp.float32),
    ],
)


def _final_body(u_ref, den_ref, res_ref, o_ref):
    i = pl.program_id(0)
    u = u_ref[0] + u_ref[1]
    dsum = den_ref[:, 0:1] + den_ref[:, 1:2]
    dsafe = jnp.where(dsum == 0.0, 1.0, dsum)
    v = u / dsafe + res_ref[...]
    h = jnp.where(v > 0, v, jnp.exp(v) - 1.0)
    part = jnp.sum(h, axis=0, keepdims=True) * (1.0 / N)

    @pl.when(i == 0)
    def _():
        o_ref[...] = jnp.zeros_like(o_ref)

    o_ref[...] += part


_final = pl.pallas_call(
    _final_body,
    grid=(GRID,),
    in_specs=[
        pl.BlockSpec((NC, ROW_BLK, D), lambda i: (0, i, 0)),
        pl.BlockSpec((ROW_BLK, NC), lambda i: (i, 0)),
        pl.BlockSpec((ROW_BLK, D), lambda i: (i, 0)),
    ],
    out_specs=pl.BlockSpec((1, D), lambda i: (0, 0)),
    out_shape=jax.ShapeDtypeStruct((1, D), jnp.float32),
)


def kernel(x, edge_index, W0, a0, W1, a1, W2, a2):
    src = edge_index[0].astype(jnp.int32)
    dst = edge_index[1].astype(jnp.int32)
    pad = jnp.zeros((E_PAD - E,), jnp.int32)
    src2d = jnp.concatenate([src, pad]).reshape(NW * NCHUNK, CHUNK)
    dst2d = jnp.concatenate([dst, pad]).reshape(NW * NCHUNK, CHUNK)
    z2 = jnp.zeros((NP, D), jnp.float32)
    z1 = jnp.zeros((NP,), jnp.float32)
    zres = jnp.zeros((NP, D), jnp.float32)
    xp = jnp.concatenate([x, jnp.zeros((NP - N, D), jnp.float32)])

    ft = _mm(xp, W0)
    u, den = _sc_edge(ft, src2d, dst2d, a0.reshape(D), z2, z1)
    h1, ft = _combine(u, den.T, zres, W1)
    u, den = _sc_edge(ft, src2d, dst2d, a1.reshape(D), z2, z1)
    h2, ft = _combine(u, den.T, h1, W2)
    u, den = _sc_edge(ft, src2d, dst2d, a2.reshape(D), z2, z1)
    return _final(u, den.T, h2)


# speedup vs baseline: 3.7068x; 1.2743x over previous
import functools

import jax
import jax.numpy as jnp
from jax import lax
from jax.experimental import pallas as pl
from jax.experimental.pallas import tpu as pltpu
from jax.experimental.pallas import tpu_sc as plsc

N = 10000
NP = 10112
D = 128
E = 320000

NC = 2
NS = 16
L = 16
NW = NC * NS

CHUNK = 64
EPW = 10240
E_PAD = NW * EPW
NCHUNK = EPW // CHUNK
IDX_BLK = 16
NBLK = NCHUNK // IDX_BLK
RPT = NP // NS
NPD = 10240
RPTD = NPD // NS

ROW_BLK = 1264
GRID = NP // ROW_BLK


def _sc_edge_body(ft_hbm, src_hbm, dst_hbm, a_hbm, z2_hbm, z1_hbm,
                  u_out, den_out,
                  src_idx, dst_idx, rows_s, rows_d, ex_v, a_v,
                  u_sh, den_sh,
                  sem_gs, sem_gd, sem_u, sem_den, sem_is, sem_id):
    c = lax.axis_index("c")
    s = lax.axis_index("s")
    w = c * NS + s

    pltpu.sync_copy(z2_hbm.at[pl.ds(s * RPT, RPT)],
                    u_sh.at[pl.ds(s * RPT, RPT)])
    pltpu.sync_copy(z1_hbm.at[pl.ds(s * RPTD, RPTD)],
                    den_sh.at[pl.ds(s * RPTD, RPTD)])
    pltpu.sync_copy(a_hbm, a_v)
    avs = [a_v[pl.ds(jd * L, L)] for jd in range(D // L)]
    lane = lax.iota(jnp.int32, L)
    lane0 = lane == 0
    splat15 = jnp.full((L,), L - 1, jnp.int32)
    plsc.subcore_barrier()

    def idx_row(buf, jg):
        b = jg // IDX_BLK
        return buf.at[lax.rem(b, 2), jg - b * IDX_BLK]

    def issue_gathers(jn):
        pltpu.async_copy(ft_hbm.at[idx_row(src_idx, jn)],
                         rows_s.at[lax.rem(jn, 2)],
                         sem_gs.at[lax.rem(jn, 2)])
        pltpu.async_copy(ft_hbm.at[idx_row(dst_idx, jn)],
                         rows_d.at[lax.rem(jn, 3)],
                         sem_gd.at[lax.rem(jn, 3)])

    pltpu.sync_copy(src_hbm.at[pl.ds(w * NCHUNK, IDX_BLK)], src_idx.at[0])
    pltpu.sync_copy(dst_hbm.at[pl.ds(w * NCHUNK, IDX_BLK)], dst_idx.at[0])
    pltpu.async_copy(src_hbm.at[pl.ds(w * NCHUNK + IDX_BLK, IDX_BLK)],
                     src_idx.at[1], sem_is)
    pltpu.async_copy(dst_hbm.at[pl.ds(w * NCHUNK + IDX_BLK, IDX_BLK)],
                     dst_idx.at[1], sem_id)
    issue_gathers(0)
    issue_gathers(1)

    def chunk_body(j, carry):
        slot2 = lax.rem(j, 2)
        slot3 = lax.rem(j, 3)
        b = j // IDX_BLK
        jj = j - b * IDX_BLK

        @pl.when(jnp.logical_and(jj == 0,
                                 jnp.logical_and(b >= 1, b < NBLK - 1)))
        def _():
            off = w * NCHUNK + (b + 1) * IDX_BLK
            sl = lax.rem(b + 1, 2)
            pltpu.async_copy(src_hbm.at[pl.ds(off, IDX_BLK)],
                             src_idx.at[sl], sem_is)
            pltpu.async_copy(dst_hbm.at[pl.ds(off, IDX_BLK)],
                             dst_idx.at[sl], sem_id)

        @pl.when(jnp.logical_and(jj == IDX_BLK - 2, b < NBLK - 1))
        def _():
            pltpu.make_async_copy(src_hbm.at[pl.ds(0, IDX_BLK)],
                                  src_idx.at[0], sem_is).wait()
            pltpu.make_async_copy(dst_hbm.at[pl.ds(0, IDX_BLK)],
                                  dst_idx.at[0], sem_id).wait()

        pltpu.make_async_copy(ft_hbm.at[pl.ds(0, CHUNK)],
                              rows_s.at[slot2], sem_gs.at[slot2]).wait()
        pltpu.make_async_copy(ft_hbm.at[pl.ds(0, CHUNK)],
                              rows_d.at[slot3], sem_gd.at[slot3]).wait()
        @pl.when(j >= 2)
        def _():
            pltpu.make_async_copy(ex_v.at[slot2],
                                  den_sh.at[pl.ds(0, CHUNK)],
                                  sem_den.at[slot2]).wait()

        ebase = w * EPW + j * CHUNK

        def edge_body(e, carry_e):
            srow = [rows_s[slot2, e, pl.ds(jd * L, L)] for jd in range(D // L)]
            drow = [rows_d[slot3, e, pl.ds(jd * L, L)] for jd in range(D // L)]
            acc = jnp.zeros((L,), jnp.float32)
            for jd in range(D // L):
                t = srow[jd] + drow[jd]
                lr = jnp.maximum(t, 0.2 * t)
                acc = acc + avs[jd] * lr
            cum = plsc.cumsum(acc)
            tot = cum.at[splat15].get(mode="promise_in_bounds")
            ex = jnp.exp(tot)
            ex = jnp.where(ebase + e < E, ex, 0.0)
            plsc.store_scatter(ex_v,
                               [jnp.full((L,), slot2, jnp.int32),
                                jnp.full((L,), e, jnp.int32)], ex,
                               mask=lane0)
            for jd in range(D // L):
                rows_d[slot3, e, pl.ds(jd * L, L)] = srow[jd] * ex
            return carry_e

        lax.fori_loop(0, CHUNK, edge_body, 0, unroll=2)

        pltpu.async_copy(rows_d.at[slot3], u_sh.at[idx_row(dst_idx, j)],
                         sem_u.at[slot3], add=True)
        pltpu.async_copy(ex_v.at[slot2], den_sh.at[idx_row(dst_idx, j)],
                         sem_den.at[slot2], add=True)

        jn = j + 2

        @pl.when(jn < NCHUNK)
        def _():
            pltpu.async_copy(ft_hbm.at[idx_row(src_idx, jn)],
                             rows_s.at[lax.rem(jn, 2)],
                             sem_gs.at[lax.rem(jn, 2)])
            @pl.when(j >= 1)
            def _():
                pltpu.make_async_copy(rows_d.at[lax.rem(jn, 3)],
                                      u_sh.at[pl.ds(0, CHUNK)],
                                      sem_u.at[lax.rem(jn, 3)]).wait()
            pltpu.async_copy(ft_hbm.at[idx_row(dst_idx, jn)],
                             rows_d.at[lax.rem(jn, 3)],
                             sem_gd.at[lax.rem(jn, 3)])

        return carry

    lax.fori_loop(0, NCHUNK, chunk_body, 0)

    for t in range(3):
        sl = (NCHUNK - 3 + t) % 3
        pltpu.make_async_copy(rows_d.at[sl], u_sh.at[pl.ds(0, CHUNK)],
                              sem_u.at[sl]).wait()
    for t in range(2):
        sl = (NCHUNK - 2 + t) % 2
        pltpu.make_async_copy(ex_v.at[sl], den_sh.at[pl.ds(0, CHUNK)],
                              sem_den.at[sl]).wait()

    plsc.subcore_barrier()
    pltpu.sync_copy(u_sh.at[pl.ds(s * RPT, RPT)],
                    u_out.at[pl.ds(c * NP + s * RPT, RPT)])
    pltpu.sync_copy(den_sh.at[pl.ds(s * RPTD, RPTD)],
                    den_out.at[pl.ds(c * NPD + s * RPTD, RPTD)])


_sc_edge = pl.kernel(
    _sc_edge_body,
    out_type=[
        jax.ShapeDtypeStruct((NC * NP, D), jnp.float32),
        jax.ShapeDtypeStruct((NC * NPD,), jnp.float32),
    ],
    mesh=plsc.VectorSubcoreMesh(core_axis_name="c", subcore_axis_name="s"),
    compiler_params=pltpu.CompilerParams(needs_layout_passes=False),
    scratch_types=[
        pltpu.VMEM((2, IDX_BLK, CHUNK), jnp.int32),
        pltpu.VMEM((2, IDX_BLK, CHUNK), jnp.int32),
        pltpu.VMEM((2, CHUNK, D), jnp.float32),
        pltpu.VMEM((3, CHUNK, D), jnp.float32),
        pltpu.VMEM((2, CHUNK), jnp.float32),
        pltpu.VMEM((D,), jnp.float32),
        pltpu.VMEM_SHARED((NP, D), jnp.float32),
        pltpu.VMEM_SHARED((NPD,), jnp.float32),
        pltpu.SemaphoreType.DMA((2,)),
        pltpu.SemaphoreType.DMA((3,)),
        pltpu.SemaphoreType.DMA((3,)),
        pltpu.SemaphoreType.DMA((2,)),
        pltpu.SemaphoreType.DMA,
        pltpu.SemaphoreType.DMA,
    ],
)


def _mm_body(x_ref, w_ref, o_ref):
    o_ref[...] = jnp.dot(x_ref[...], w_ref[...],
                         preferred_element_type=jnp.float32)


_mm = pl.pallas_call(
    _mm_body,
    grid=(GRID,),
    in_specs=[
        pl.BlockSpec((ROW_BLK, D), lambda i: (i, 0)),
        pl.BlockSpec((D, D), lambda i: (0, 0)),
    ],
    out_specs=pl.BlockSpec((ROW_BLK, D), lambda i: (i, 0)),
    out_shape=jax.ShapeDtypeStruct((NP, D), jnp.float32),
)


def _combine_body(u_ref, den_ref, res_ref, w_ref, h_ref, ft_ref):
    u = u_ref[0] + u_ref[1]
    dsum = den_ref[:, 0:1] + den_ref[:, 1:2]
    dsafe = jnp.where(dsum == 0.0, 1.0, dsum)
    v = u / dsafe + res_ref[...]
    h = jnp.where(v > 0, v, jnp.exp(v) - 1.0)
    h_ref[...] = h
    ft_ref[...] = jnp.dot(h, w_ref[...], preferred_element_type=jnp.float32)


_combine = pl.pallas_call(
    _combine_body,
    grid=(GRID,),
    in_specs=[
        pl.BlockSpec((NC, ROW_BLK, D), lambda i: (0, i, 0)),
        pl.BlockSpec((ROW_BLK, NC), lambda i: (i, 0)),
        pl.BlockSpec((ROW_BLK, D), lambda i: (i, 0)),
        pl.BlockSpec((D, D), lambda i: (0, 0)),
    ],
    out_specs=[
        pl.BlockSpec((ROW_BLK, D), lambda i: (i, 0)),
        pl.BlockSpec((ROW_BLK, D), lambda i: (i, 0)),
    ],
    out_shape=[
        jax.ShapeDtypeStruct((NP, D), jnp.float32),
        jax.ShapeDtypeStruct((NP, D), jnp.float32),
    ],
)


def _final_body(u_ref, den_ref, res_ref, o_ref):
    i = pl.program_id(0)
    u = u_ref[0] + u_ref[1]
    dsum = den_ref[:, 0:1] + den_ref[:, 1:2]
    dsafe = jnp.where(dsum == 0.0, 1.0, dsum)
    v = u / dsafe + res_ref[...]
    h = jnp.where(v > 0, v, jnp.exp(v) - 1.0)
    part = jnp.sum(h, axis=0, keepdims=True) * (1.0 / N)

    @pl.when(i == 0)
    def _():
        o_ref[...] = jnp.zeros_like(o_ref)

    o_ref[...] += part


_final = pl.pallas_call(
    _final_body,
    grid=(GRID,),
    in_specs=[
        pl.BlockSpec((NC, ROW_BLK, D), lambda i: (0, i, 0)),
        pl.BlockSpec((ROW_BLK, NC), lambda i: (i, 0)),
        pl.BlockSpec((ROW_BLK, D), lambda i: (i, 0)),
    ],
    out_specs=pl.BlockSpec((1, D), lambda i: (0, 0)),
    out_shape=jax.ShapeDtypeStruct((1, D), jnp.float32),
)


def kernel(x, edge_index, W0, a0, W1, a1, W2, a2):
    src = edge_index[0].astype(jnp.int32)
    dst = edge_index[1].astype(jnp.int32)
    pad = jnp.zeros((E_PAD - E,), jnp.int32)
    src2d = jnp.concatenate([src, pad]).reshape(NW * NCHUNK, CHUNK)
    dst2d = jnp.concatenate([dst, pad]).reshape(NW * NCHUNK, CHUNK)
    z2 = jnp.zeros((NP, D), jnp.float32)
    z1 = jnp.zeros((NPD,), jnp.float32)
    zres = jnp.zeros((NP, D), jnp.float32)
    xp = jnp.concatenate([x, jnp.zeros((NP - N, D), jnp.float32)])

    ft = _mm(xp, W0)
    u, den = _sc_edge(ft, src2d, dst2d, a0.reshape(D), z2, z1)
    h1, ft = _combine(u.reshape(NC, NP, D), den.reshape(NC, NPD)[:, :NP].T, zres, W1)
    u, den = _sc_edge(ft, src2d, dst2d, a1.reshape(D), z2, z1)
    h2, ft = _combine(u.reshape(NC, NP, D), den.reshape(NC, NPD)[:, :NP].T, h1, W2)
    u, den = _sc_edge(ft, src2d, dst2d, a2.reshape(D), z2, z1)
    return _final(u.reshape(NC, NP, D), den.reshape(NC, NPD)[:, :NP].T, h2)


# speedup vs baseline: 6.2349x; 1.6820x over previous
import functools

import jax
import jax.numpy as jnp
from jax import lax
from jax.experimental import pallas as pl
from jax.experimental.pallas import tpu as pltpu
from jax.experimental.pallas import tpu_sc as plsc

N = 10000
NP = 10112
D = 128
E = 320000

NC = 2
NS = 16
L = 16
NW = NC * NS

CHUNK = 64
EPW = 10240
E_PAD = NW * EPW
NCHUNK = EPW // CHUNK
IDX_BLK = 16
NBLK = NCHUNK // IDX_BLK
RPT = NP // NS
NPD = 10240
RPTD = NPD // NS

ROW_BLK = 1264
GRID = NP // ROW_BLK


def _sc_edge_body(ft_hbm, src_hbm, dst_hbm, a_hbm, z2_hbm, z1_hbm,
                  u_out, den_out,
                  src_idx, dst_idx, rows_s, rows_d, ex_v, a_v,
                  u_sh, den_sh,
                  sem_gs, sem_gd, sem_u, sem_den, sem_is, sem_id):
    c = lax.axis_index("c")
    s = lax.axis_index("s")
    w = c * NS + s

    pltpu.sync_copy(z2_hbm.at[pl.ds(s * RPT, RPT)],
                    u_sh.at[pl.ds(s * RPT, RPT)])
    pltpu.sync_copy(z1_hbm.at[pl.ds(s * RPTD, RPTD)],
                    den_sh.at[pl.ds(s * RPTD, RPTD)])
    pltpu.sync_copy(a_hbm, a_v)
    avs = [a_v[pl.ds(jd * L, L)] for jd in range(D // L)]
    lane = lax.iota(jnp.int32, L)
    lane0 = lane == 0
    splat15 = jnp.full((L,), L - 1, jnp.int32)
    plsc.subcore_barrier()

    def idx_row(buf, jg):
        b = jg // IDX_BLK
        return buf.at[lax.rem(b, 2), jg - b * IDX_BLK]

    def issue_gathers(jn):
        pltpu.async_copy(ft_hbm.at[idx_row(src_idx, jn)],
                         rows_s.at[lax.rem(jn, 2)],
                         sem_gs.at[lax.rem(jn, 2)])
        pltpu.async_copy(ft_hbm.at[idx_row(dst_idx, jn)],
                         rows_d.at[lax.rem(jn, 3)],
                         sem_gd.at[lax.rem(jn, 3)])

    pltpu.sync_copy(src_hbm.at[pl.ds(w * NCHUNK, IDX_BLK)], src_idx.at[0])
    pltpu.sync_copy(dst_hbm.at[pl.ds(w * NCHUNK, IDX_BLK)], dst_idx.at[0])
    pltpu.async_copy(src_hbm.at[pl.ds(w * NCHUNK + IDX_BLK, IDX_BLK)],
                     src_idx.at[1], sem_is)
    pltpu.async_copy(dst_hbm.at[pl.ds(w * NCHUNK + IDX_BLK, IDX_BLK)],
                     dst_idx.at[1], sem_id)
    issue_gathers(0)
    issue_gathers(1)

    def chunk_body(j, carry):
        slot2 = lax.rem(j, 2)
        slot3 = lax.rem(j, 3)
        b = j // IDX_BLK
        jj = j - b * IDX_BLK

        @pl.when(jnp.logical_and(jj == 0,
                                 jnp.logical_and(b >= 1, b < NBLK - 1)))
        def _():
            off = w * NCHUNK + (b + 1) * IDX_BLK
            sl = lax.rem(b + 1, 2)
            pltpu.async_copy(src_hbm.at[pl.ds(off, IDX_BLK)],
                             src_idx.at[sl], sem_is)
            pltpu.async_copy(dst_hbm.at[pl.ds(off, IDX_BLK)],
                             dst_idx.at[sl], sem_id)

        @pl.when(jnp.logical_and(jj == IDX_BLK - 2, b < NBLK - 1))
        def _():
            pltpu.make_async_copy(src_hbm.at[pl.ds(0, IDX_BLK)],
                                  src_idx.at[0], sem_is).wait()
            pltpu.make_async_copy(dst_hbm.at[pl.ds(0, IDX_BLK)],
                                  dst_idx.at[0], sem_id).wait()

        pltpu.make_async_copy(ft_hbm.at[pl.ds(0, CHUNK)],
                              rows_s.at[slot2], sem_gs.at[slot2]).wait()
        pltpu.make_async_copy(ft_hbm.at[pl.ds(0, CHUNK)],
                              rows_d.at[slot3], sem_gd.at[slot3]).wait()
        @pl.when(j >= 2)
        def _():
            pltpu.make_async_copy(ex_v.at[slot2],
                                  den_sh.at[pl.ds(0, CHUNK)],
                                  sem_den.at[slot2]).wait()

        ebase = w * EPW + j * CHUNK

        def edge_body(e, carry_e):
            srow = [rows_s[slot2, e, pl.ds(jd * L, L)] for jd in range(D // L)]
            drow = [rows_d[slot3, e, pl.ds(jd * L, L)] for jd in range(D // L)]
            acc = jnp.zeros((L,), jnp.float32)
            for jd in range(D // L):
                t = srow[jd] + drow[jd]
                lr = jnp.maximum(t, 0.2 * t)
                acc = acc + avs[jd] * lr
            cum = plsc.cumsum(acc)
            tot = cum.at[splat15].get(mode="promise_in_bounds")
            ex = jnp.exp(tot)
            ex = jnp.where(ebase + e < E, ex, 0.0)
            plsc.store_scatter(ex_v,
                               [jnp.full((L,), slot2, jnp.int32),
                                jnp.full((L,), e, jnp.int32)], ex,
                               mask=lane0)
            for jd in range(D // L):
                rows_d[slot3, e, pl.ds(jd * L, L)] = srow[jd] * ex
            return carry_e

        lax.fori_loop(0, CHUNK, edge_body, 0, unroll=2)

        pltpu.async_copy(rows_d.at[slot3], u_sh.at[idx_row(dst_idx, j)],
                         sem_u.at[slot3], add=True)
        pltpu.async_copy(ex_v.at[slot2], den_sh.at[idx_row(dst_idx, j)],
                         sem_den.at[slot2], add=True)

        jn = j + 2

        @pl.when(jn < NCHUNK)
        def _():
            pltpu.async_copy(ft_hbm.at[idx_row(src_idx, jn)],
                             rows_s.at[lax.rem(jn, 2)],
                             sem_gs.at[lax.rem(jn, 2)])
            @pl.when(j >= 1)
            def _():
                pltpu.make_async_copy(rows_d.at[lax.rem(jn, 3)],
                                      u_sh.at[pl.ds(0, CHUNK)],
                                      sem_u.at[lax.rem(jn, 3)]).wait()
            pltpu.async_copy(ft_hbm.at[idx_row(dst_idx, jn)],
                             rows_d.at[lax.rem(jn, 3)],
                             sem_gd.at[lax.rem(jn, 3)])

        return carry

    lax.fori_loop(0, NCHUNK, chunk_body, 0)

    for t in range(3):
        sl = (NCHUNK - 3 + t) % 3
        pltpu.make_async_copy(rows_d.at[sl], u_sh.at[pl.ds(0, CHUNK)],
                              sem_u.at[sl]).wait()
    for t in range(2):
        sl = (NCHUNK - 2 + t) % 2
        pltpu.make_async_copy(ex_v.at[sl], den_sh.at[pl.ds(0, CHUNK)],
                              sem_den.at[sl]).wait()

    plsc.subcore_barrier()
    pltpu.sync_copy(u_sh.at[pl.ds(s * RPT, RPT)],
                    u_out.at[pl.ds(c * NP + s * RPT, RPT)])
    pltpu.sync_copy(den_sh.at[pl.ds(s * RPTD, RPTD)],
                    den_out.at[pl.ds(c * NPD + s * RPTD, RPTD)])


_sc_edge = pl.kernel(
    _sc_edge_body,
    out_type=[
        jax.ShapeDtypeStruct((NC * NP, D), jnp.float32),
        jax.ShapeDtypeStruct((NC * NPD,), jnp.float32),
    ],
    mesh=plsc.VectorSubcoreMesh(core_axis_name="c", subcore_axis_name="s"),
    compiler_params=pltpu.CompilerParams(needs_layout_passes=False),
    scratch_types=[
        pltpu.VMEM((2, IDX_BLK, CHUNK), jnp.int32),
        pltpu.VMEM((2, IDX_BLK, CHUNK), jnp.int32),
        pltpu.VMEM((2, CHUNK, D), jnp.float32),
        pltpu.VMEM((3, CHUNK, D), jnp.float32),
        pltpu.VMEM((2, CHUNK), jnp.float32),
        pltpu.VMEM((D,), jnp.float32),
        pltpu.VMEM_SHARED((NP, D), jnp.float32),
        pltpu.VMEM_SHARED((NPD,), jnp.float32),
        pltpu.SemaphoreType.DMA((2,)),
        pltpu.SemaphoreType.DMA((3,)),
        pltpu.SemaphoreType.DMA((3,)),
        pltpu.SemaphoreType.DMA((2,)),
        pltpu.SemaphoreType.DMA,
        pltpu.SemaphoreType.DMA,
    ],
)


def _mm_body(x_ref, w_ref, o_ref):
    o_ref[...] = jnp.dot(x_ref[...], w_ref[...],
                         preferred_element_type=jnp.float32)


_mm = pl.pallas_call(
    _mm_body,
    grid=(GRID,),
    in_specs=[
        pl.BlockSpec((ROW_BLK, D), lambda i: (i, 0)),
        pl.BlockSpec((D, D), lambda i: (0, 0)),
    ],
    out_specs=pl.BlockSpec((ROW_BLK, D), lambda i: (i, 0)),
    out_shape=jax.ShapeDtypeStruct((NP, D), jnp.float32),
)


def _combine_body(u_ref, den_ref, res_ref, w_ref, h_ref, ft_ref):
    u = u_ref[0] + u_ref[1]
    dsum = den_ref[:, 0:1] + den_ref[:, 1:2]
    dsafe = jnp.where(dsum == 0.0, 1.0, dsum)
    v = u / dsafe + res_ref[...]
    h = jnp.where(v > 0, v, jnp.exp(v) - 1.0)
    h_ref[...] = h
    ft_ref[...] = jnp.dot(h, w_ref[...], preferred_element_type=jnp.float32)


_combine = pl.pallas_call(
    _combine_body,
    grid=(GRID,),
    in_specs=[
        pl.BlockSpec((NC, ROW_BLK, D), lambda i: (0, i, 0)),
        pl.BlockSpec((ROW_BLK, NC), lambda i: (i, 0)),
        pl.BlockSpec((ROW_BLK, D), lambda i: (i, 0)),
        pl.BlockSpec((D, D), lambda i: (0, 0)),
    ],
    out_specs=[
        pl.BlockSpec((ROW_BLK, D), lambda i: (i, 0)),
        pl.BlockSpec((ROW_BLK, D), lambda i: (i, 0)),
    ],
    out_shape=[
        jax.ShapeDtypeStruct((NP, D), jnp.float32),
        jax.ShapeDtypeStruct((NP, D), jnp.float32),
    ],
)


def _final_body(u_ref, den_ref, res_ref, o_ref):
    i = pl.program_id(0)
    u = u_ref[0] + u_ref[1]
    dsum = den_ref[:, 0:1] + den_ref[:, 1:2]
    dsafe = jnp.where(dsum == 0.0, 1.0, dsum)
    v = u / dsafe + res_ref[...]
    h = jnp.where(v > 0, v, jnp.exp(v) - 1.0)
    part = jnp.sum(h, axis=0, keepdims=True) * (1.0 / N)

    @pl.when(i == 0)
    def _():
        o_ref[...] = jnp.zeros_like(o_ref)

    o_ref[...] += part


_final = pl.pallas_call(
    _final_body,
    grid=(GRID,),
    in_specs=[
        pl.BlockSpec((NC, ROW_BLK, D), lambda i: (0, i, 0)),
        pl.BlockSpec((ROW_BLK, NC), lambda i: (i, 0)),
        pl.BlockSpec((ROW_BLK, D), lambda i: (i, 0)),
    ],
    out_specs=pl.BlockSpec((1, D), lambda i: (0, 0)),
    out_shape=jax.ShapeDtypeStruct((1, D), jnp.float32),
)


def kernel(x, edge_index, W0, a0, W1, a1, W2, a2):
    src = edge_index[0].astype(jnp.int32)
    dst = edge_index[1].astype(jnp.int32)
    pad_src = jnp.zeros((E_PAD - E,), jnp.int32)
    pad_dst = N + (jnp.arange(E_PAD - E, dtype=jnp.int32) % (NP - N))
    src2d = jnp.concatenate([src, pad_src]).reshape(NW * NCHUNK, CHUNK)
    dst2d = jnp.concatenate([dst, pad_dst]).reshape(NW * NCHUNK, CHUNK)
    z2 = jnp.zeros((NP, D), jnp.float32)
    z1 = jnp.zeros((NPD,), jnp.float32)
    zres = jnp.zeros((NP, D), jnp.float32)
    xp = jnp.concatenate([x, jnp.zeros((NP - N, D), jnp.float32)])

    ft = _mm(xp, W0)
    u, den = _sc_edge(ft, src2d, dst2d, a0.reshape(D), z2, z1)
    h1, ft = _combine(u.reshape(NC, NP, D), den.reshape(NC, NPD)[:, :NP].T, zres, W1)
    u, den = _sc_edge(ft, src2d, dst2d, a1.reshape(D), z2, z1)
    h2, ft = _combine(u.reshape(NC, NP, D), den.reshape(NC, NPD)[:, :NP].T, h1, W2)
    u, den = _sc_edge(ft, src2d, dst2d, a2.reshape(D), z2, z1)
    return _final(u.reshape(NC, NP, D), den.reshape(NC, NPD)[:, :NP].T, h2)


# speedup vs baseline: 8.8636x; 1.4216x over previous
import functools

import jax
import jax.numpy as jnp
from jax import lax
from jax.experimental import pallas as pl
from jax.experimental.pallas import tpu as pltpu
from jax.experimental.pallas import tpu_sc as plsc

N = 10000
NP = 10112
D = 128
E = 320000

NC = 2
NS = 16
L = 16
NW = NC * NS

CHUNK = 64
EPW = 10240
E_PAD = NW * EPW
NCHUNK = EPW // CHUNK
IDX_BLK = 16
NBLK = NCHUNK // IDX_BLK
RPT = NP // NS
NPD = 10240
RPTD = NPD // NS

ROW_BLK = 1264
GRID = NP // ROW_BLK


def _sc_edge_body(ft_hbm, src_hbm, dst_hbm, a_hbm, z2_hbm, z1_hbm,
                  u_out, den_out,
                  src_idx, dst_idx, rows_s, rows_d, ex_v, a_v,
                  u_sh, den_sh,
                  sem_gs, sem_gd, sem_u, sem_den, sem_is, sem_id):
    c = lax.axis_index("c")
    s = lax.axis_index("s")
    w = c * NS + s

    pltpu.sync_copy(z2_hbm.at[pl.ds(s * RPT, RPT)],
                    u_sh.at[pl.ds(s * RPT, RPT)])
    pltpu.sync_copy(z1_hbm.at[pl.ds(s * RPTD, RPTD)],
                    den_sh.at[pl.ds(s * RPTD, RPTD)])
    pltpu.sync_copy(a_hbm, a_v)
    avs = [a_v[pl.ds(jd * L, L)] for jd in range(D // L)]
    lane = lax.iota(jnp.int32, L)
    lane0 = lane == 0
    splat15 = jnp.full((L,), L - 1, jnp.int32)
    plsc.subcore_barrier()

    def idx_row(buf, jg):
        b = jg // IDX_BLK
        return buf.at[lax.rem(b, 2), jg - b * IDX_BLK]

    def issue_gathers(jn):
        pltpu.async_copy(ft_hbm.at[idx_row(src_idx, jn)],
                         rows_s.at[lax.rem(jn, 2)],
                         sem_gs.at[lax.rem(jn, 2)])
        pltpu.async_copy(ft_hbm.at[idx_row(dst_idx, jn)],
                         rows_d.at[lax.rem(jn, 3)],
                         sem_gd.at[lax.rem(jn, 3)])

    pltpu.sync_copy(src_hbm.at[pl.ds(w * NCHUNK, IDX_BLK)], src_idx.at[0])
    pltpu.sync_copy(dst_hbm.at[pl.ds(w * NCHUNK, IDX_BLK)], dst_idx.at[0])
    pltpu.async_copy(src_hbm.at[pl.ds(w * NCHUNK + IDX_BLK, IDX_BLK)],
                     src_idx.at[1], sem_is)
    pltpu.async_copy(dst_hbm.at[pl.ds(w * NCHUNK + IDX_BLK, IDX_BLK)],
                     dst_idx.at[1], sem_id)
    issue_gathers(0)
    issue_gathers(1)

    def chunk_body(j, carry):
        slot2 = lax.rem(j, 2)
        slot3 = lax.rem(j, 3)
        b = j // IDX_BLK
        jj = j - b * IDX_BLK

        @pl.when(jnp.logical_and(jj == 0,
                                 jnp.logical_and(b >= 1, b < NBLK - 1)))
        def _():
            off = w * NCHUNK + (b + 1) * IDX_BLK
            sl = lax.rem(b + 1, 2)
            pltpu.async_copy(src_hbm.at[pl.ds(off, IDX_BLK)],
                             src_idx.at[sl], sem_is)
            pltpu.async_copy(dst_hbm.at[pl.ds(off, IDX_BLK)],
                             dst_idx.at[sl], sem_id)

        @pl.when(jnp.logical_and(jj == IDX_BLK - 2, b < NBLK - 1))
        def _():
            pltpu.make_async_copy(src_hbm.at[pl.ds(0, IDX_BLK)],
                                  src_idx.at[0], sem_is).wait()
            pltpu.make_async_copy(dst_hbm.at[pl.ds(0, IDX_BLK)],
                                  dst_idx.at[0], sem_id).wait()

        ebase = w * EPW + j * CHUNK
        real_j = ebase < E

        @pl.when(real_j)
        def _():
            pltpu.make_async_copy(ft_hbm.at[pl.ds(0, CHUNK)],
                                  rows_s.at[slot2], sem_gs.at[slot2]).wait()
            pltpu.make_async_copy(ft_hbm.at[pl.ds(0, CHUNK)],
                                  rows_d.at[slot3], sem_gd.at[slot3]).wait()
            @pl.when(j >= 2)
            def _():
                pltpu.make_async_copy(ex_v.at[slot2],
                                      den_sh.at[pl.ds(0, CHUNK)],
                                      sem_den.at[slot2]).wait()

            def edge_body(e, carry_e):
                srow = [rows_s[slot2, e, pl.ds(jd * L, L)]
                        for jd in range(D // L)]
                drow = [rows_d[slot3, e, pl.ds(jd * L, L)]
                        for jd in range(D // L)]
                acc = jnp.zeros((L,), jnp.float32)
                for jd in range(D // L):
                    t = srow[jd] + drow[jd]
                    lr = jnp.maximum(t, 0.2 * t)
                    acc = acc + avs[jd] * lr
                cum = plsc.cumsum(acc)
                tot = cum.at[splat15].get(mode="promise_in_bounds")
                ex = jnp.exp(tot)
                plsc.store_scatter(ex_v,
                                   [jnp.full((L,), slot2, jnp.int32),
                                    jnp.full((L,), e, jnp.int32)], ex,
                                   mask=lane0)
                for jd in range(D // L):
                    rows_d[slot3, e, pl.ds(jd * L, L)] = srow[jd] * ex
                return carry_e

            lax.fori_loop(0, CHUNK, edge_body, 0, unroll=2)

            pltpu.async_copy(rows_d.at[slot3], u_sh.at[idx_row(dst_idx, j)],
                             sem_u.at[slot3], add=True)
            pltpu.async_copy(ex_v.at[slot2], den_sh.at[idx_row(dst_idx, j)],
                             sem_den.at[slot2], add=True)

        jn = j + 2

        @pl.when(jnp.logical_and(jn < NCHUNK, w * EPW + jn * CHUNK < E))
        def _():
            pltpu.async_copy(ft_hbm.at[idx_row(src_idx, jn)],
                             rows_s.at[lax.rem(jn, 2)],
                             sem_gs.at[lax.rem(jn, 2)])
            @pl.when(j >= 1)
            def _():
                pltpu.make_async_copy(rows_d.at[lax.rem(jn, 3)],
                                      u_sh.at[pl.ds(0, CHUNK)],
                                      sem_u.at[lax.rem(jn, 3)]).wait()
            pltpu.async_copy(ft_hbm.at[idx_row(dst_idx, jn)],
                             rows_d.at[lax.rem(jn, 3)],
                             sem_gd.at[lax.rem(jn, 3)])

        return carry

    lax.fori_loop(0, NCHUNK, chunk_body, 0)

    for t in range(3):
        sl = (NCHUNK - 3 + t) % 3
        pltpu.make_async_copy(rows_d.at[sl], u_sh.at[pl.ds(0, CHUNK)],
                              sem_u.at[sl]).wait()
    for t in range(2):
        sl = (NCHUNK - 2 + t) % 2
        pltpu.make_async_copy(ex_v.at[sl], den_sh.at[pl.ds(0, CHUNK)],
                              sem_den.at[sl]).wait()

    plsc.subcore_barrier()
    pltpu.sync_copy(u_sh.at[pl.ds(s * RPT, RPT)],
                    u_out.at[pl.ds(c * NP + s * RPT, RPT)])
    pltpu.sync_copy(den_sh.at[pl.ds(s * RPTD, RPTD)],
                    den_out.at[pl.ds(c * NPD + s * RPTD, RPTD)])


_sc_edge = pl.kernel(
    _sc_edge_body,
    out_type=[
        jax.ShapeDtypeStruct((NC * NP, D), jnp.float32),
        jax.ShapeDtypeStruct((NC * NPD,), jnp.float32),
    ],
    mesh=plsc.VectorSubcoreMesh(core_axis_name="c", subcore_axis_name="s"),
    compiler_params=pltpu.CompilerParams(needs_layout_passes=False),
    scratch_types=[
        pltpu.VMEM((2, IDX_BLK, CHUNK), jnp.int32),
        pltpu.VMEM((2, IDX_BLK, CHUNK), jnp.int32),
        pltpu.VMEM((2, CHUNK, D), jnp.float32),
        pltpu.VMEM((3, CHUNK, D), jnp.float32),
        pltpu.VMEM((2, CHUNK), jnp.float32),
        pltpu.VMEM((D,), jnp.float32),
        pltpu.VMEM_SHARED((NP, D), jnp.float32),
        pltpu.VMEM_SHARED((NPD,), jnp.float32),
        pltpu.SemaphoreType.DMA((2,)),
        pltpu.SemaphoreType.DMA((3,)),
        pltpu.SemaphoreType.DMA((3,)),
        pltpu.SemaphoreType.DMA((2,)),
        pltpu.SemaphoreType.DMA,
        pltpu.SemaphoreType.DMA,
    ],
)


def _mm_body(x_ref, w_ref, o_ref):
    o_ref[...] = jnp.dot(x_ref[...], w_ref[...],
                         preferred_element_type=jnp.float32)


_mm = pl.pallas_call(
    _mm_body,
    grid=(GRID,),
    in_specs=[
        pl.BlockSpec((ROW_BLK, D), lambda i: (i, 0)),
        pl.BlockSpec((D, D), lambda i: (0, 0)),
    ],
    out_specs=pl.BlockSpec((ROW_BLK, D), lambda i: (i, 0)),
    out_shape=jax.ShapeDtypeStruct((NP, D), jnp.float32),
)


def _combine_body(u_ref, den_ref, res_ref, w_ref, h_ref, ft_ref):
    u = u_ref[0] + u_ref[1]
    dsum = den_ref[:, 0:1] + den_ref[:, 1:2]
    dsafe = jnp.where(dsum == 0.0, 1.0, dsum)
    v = u / dsafe + res_ref[...]
    h = jnp.where(v > 0, v, jnp.exp(v) - 1.0)
    h_ref[...] = h
    ft_ref[...] = jnp.dot(h, w_ref[...], preferred_element_type=jnp.float32)


_combine = pl.pallas_call(
    _combine_body,
    grid=(GRID,),
    in_specs=[
        pl.BlockSpec((NC, ROW_BLK, D), lambda i: (0, i, 0)),
        pl.BlockSpec((ROW_BLK, NC), lambda i: (i, 0)),
        pl.BlockSpec((ROW_BLK, D), lambda i: (i, 0)),
        pl.BlockSpec((D, D), lambda i: (0, 0)),
    ],
    out_specs=[
        pl.BlockSpec((ROW_BLK, D), lambda i: (i, 0)),
        pl.BlockSpec((ROW_BLK, D), lambda i: (i, 0)),
    ],
    out_shape=[
        jax.ShapeDtypeStruct((NP, D), jnp.float32),
        jax.ShapeDtypeStruct((NP, D), jnp.float32),
    ],
)


def _final_body(u_ref, den_ref, res_ref, o_ref):
    i = pl.program_id(0)
    u = u_ref[0] + u_ref[1]
    dsum = den_ref[:, 0:1] + den_ref[:, 1:2]
    dsafe = jnp.where(dsum == 0.0, 1.0, dsum)
    v = u / dsafe + res_ref[...]
    h = jnp.where(v > 0, v, jnp.exp(v) - 1.0)
    part = jnp.sum(h, axis=0, keepdims=True) * (1.0 / N)

    @pl.when(i == 0)
    def _():
        o_ref[...] = jnp.zeros_like(o_ref)

    o_ref[...] += part


_final = pl.pallas_call(
    _final_body,
    grid=(GRID,),
    in_specs=[
        pl.BlockSpec((NC, ROW_BLK, D), lambda i: (0, i, 0)),
        pl.BlockSpec((ROW_BLK, NC), lambda i: (i, 0)),
        pl.BlockSpec((ROW_BLK, D), lambda i: (i, 0)),
    ],
    out_specs=pl.BlockSpec((1, D), lambda i: (0, 0)),
    out_shape=jax.ShapeDtypeStruct((1, D), jnp.float32),
)


def kernel(x, edge_index, W0, a0, W1, a1, W2, a2):
    src = edge_index[0].astype(jnp.int32)
    dst = edge_index[1].astype(jnp.int32)
    pad_src = jnp.zeros((E_PAD - E,), jnp.int32)
    pad_dst = N + (jnp.arange(E_PAD - E, dtype=jnp.int32) % (NP - N))
    src2d = jnp.concatenate([src, pad_src]).reshape(NW * NCHUNK, CHUNK)
    dst2d = jnp.concatenate([dst, pad_dst]).reshape(NW * NCHUNK, CHUNK)
    z2 = jnp.zeros((NP, D), jnp.float32)
    z1 = jnp.zeros((NPD,), jnp.float32)
    zres = jnp.zeros((NP, D), jnp.float32)
    xp = jnp.concatenate([x, jnp.zeros((NP - N, D), jnp.float32)])

    ft = _mm(xp, W0)
    u, den = _sc_edge(ft, src2d, dst2d, a0.reshape(D), z2, z1)
    h1, ft = _combine(u.reshape(NC, NP, D), den.reshape(NC, NPD)[:, :NP].T, zres, W1)
    u, den = _sc_edge(ft, src2d, dst2d, a1.reshape(D), z2, z1)
    h2, ft = _combine(u.reshape(NC, NP, D), den.reshape(NC, NPD)[:, :NP].T, h1, W2)
    u, den = _sc_edge(ft, src2d, dst2d, a2.reshape(D), z2, z1)
    return _final(u.reshape(NC, NP, D), den.reshape(NC, NPD)[:, :NP].T, h2)
